# baseline (speedup 1.0000x reference)
import jax
import jax.numpy as jnp
from jax.experimental import pallas as pl
from jax.experimental.pallas import tpu as pltpu

HIDDEN = 1024
HEADS = 16
ATT = HIDDEN // HEADS
SCALE = ATT ** -0.5
NC = 4


def _mha_kernel(q_ref, k_ref, v_ref, wq_ref, bq_ref, wk_ref, wv_ref, bv_ref,
                wo_ref, bo_ref, out_ref, u_ref, m_ref, s_ref, at_ref):
    b = pl.program_id(0)
    c = pl.program_id(1)

    @pl.when(jnp.logical_and(b == 0, c == 0))
    def _prologue():
        qh = jax.lax.dot_general(q_ref[...], wq_ref[...],
                                 (((1,), (0,)), ((), ())),
                                 preferred_element_type=jnp.float32)
        qh = (qh + bq_ref[...]) * SCALE
        for h in range(HEADS):
            qs = qh[:, h * ATT:(h + 1) * ATT]
            ws = wk_ref[:, h * ATT:(h + 1) * ATT]
            u_ref[:, h, :] = jax.lax.dot_general(
                qs, ws, (((1,), (1,)), ((), ())),
                preferred_element_type=jnp.float32)

    @pl.when(c == 0)
    def _reset():
        m_ref[...] = jnp.full_like(m_ref, -jnp.inf)
        s_ref[...] = jnp.zeros_like(s_ref)
        at_ref[...] = jnp.zeros_like(at_ref)

    u = u_ref[b]
    kc = k_ref[0]
    scores = jax.lax.dot_general(kc, u, (((1,), (1,)), ((), ())),
                                 preferred_element_type=jnp.float32)
    m_old = m_ref[...]
    m_new = jnp.maximum(m_old, jnp.max(scores, axis=0, keepdims=True))
    alpha = jnp.exp(m_old - m_new)
    e = jnp.exp(scores - m_new)
    m_ref[...] = m_new
    s_ref[...] = s_ref[...] * alpha + jnp.sum(e, axis=0, keepdims=True)
    vc = v_ref[0]
    pv = jax.lax.dot_general(vc, e, (((0,), (0,)), ((), ())),
                             preferred_element_type=jnp.float32)
    at_ref[...] = at_ref[...] * alpha + pv

    @pl.when(c == NC - 1)
    def _epilogue():
        at = at_ref[...] * (1.0 / s_ref[...])
        f = jax.lax.dot_general(at, wv_ref[...], (((0,), (0,)), ((), ())),
                                preferred_element_type=jnp.float32)
        col_head = jax.lax.broadcasted_iota(
            jnp.int32, (HEADS, HIDDEN), 1) // ATT
        row_head = jax.lax.broadcasted_iota(
            jnp.int32, (HEADS, HIDDEN), 0)
        mask = (col_head == row_head).astype(jnp.float32)
        upd = jnp.sum(f * mask, axis=0, keepdims=True) + bv_ref[...]
        out_ref[0] = jax.lax.dot_general(
            upd, wo_ref[...], (((1,), (0,)), ((), ())),
            preferred_element_type=jnp.float32) + bo_ref[...]


def kernel(q, k, v, Wq, bq, Wk, bk, Wv, bv, Wo, bo):
    del bk
    B = q.shape[0]
    L_K = k.shape[1]
    C = L_K // NC
    full = lambda b, c: (0, 0)
    out = pl.pallas_call(
        _mha_kernel,
        grid=(B, NC),
        in_specs=[
            pl.BlockSpec((B, HIDDEN), full),
            pl.BlockSpec((1, C, HIDDEN), lambda b, c: (b, c, 0)),
            pl.BlockSpec((1, C, HIDDEN), lambda b, c: (b, c, 0)),
            pl.BlockSpec((HIDDEN, HIDDEN), full),
            pl.BlockSpec((1, HIDDEN), full),
            pl.BlockSpec((HIDDEN, HIDDEN), full),
            pl.BlockSpec((HIDDEN, HIDDEN), full),
            pl.BlockSpec((1, HIDDEN), full),
            pl.BlockSpec((HIDDEN, HIDDEN), full),
            pl.BlockSpec((1, HIDDEN), full),
        ],
        out_specs=pl.BlockSpec((1, 1, HIDDEN), lambda b, c: (b, 0, 0)),
        out_shape=jax.ShapeDtypeStruct((B, 1, HIDDEN), jnp.float32),
        scratch_shapes=[
            pltpu.VMEM((B, HEADS, HIDDEN), jnp.float32),
            pltpu.VMEM((1, HEADS), jnp.float32),
            pltpu.VMEM((1, HEADS), jnp.float32),
            pltpu.VMEM((HIDDEN, HEADS), jnp.float32),
        ],
    )(q, k, v, Wq, bq.reshape(1, HIDDEN), Wk, Wv,
      bv.reshape(1, HIDDEN), Wo, bo.reshape(1, HIDDEN))
    return out.reshape(B, HIDDEN)


# speedup vs baseline: 1.5645x; 1.5645x over previous
import jax
import jax.numpy as jnp
from jax.experimental import pallas as pl
from jax.experimental.pallas import tpu as pltpu

HIDDEN = 1024
HEADS = 16
ATT = HIDDEN // HEADS
SCALE = ATT ** -0.5


def _mha_kernel(q_ref, k_ref, v_ref, wq_ref, bq_ref, wk_ref, wv_ref, bv_ref,
                wo_ref, bo_ref, out_ref, u_ref, a_ref, s_ref, upd_ref):
    b = pl.program_id(0)
    B = q_ref.shape[0]

    @pl.when(b == 0)
    def _prologue():
        qh = jax.lax.dot_general(q_ref[...], wq_ref[...],
                                 (((1,), (0,)), ((), ())),
                                 preferred_element_type=jnp.float32)
        qh = (qh + bq_ref[...]) * SCALE
        for h in range(HEADS):
            qs = qh[:, h * ATT:(h + 1) * ATT]
            ws = wk_ref[:, h * ATT:(h + 1) * ATT]
            u_ref[:, h, :] = jax.lax.dot_general(
                qs, ws, (((1,), (1,)), ((), ())),
                preferred_element_type=jnp.float32)

    u = u_ref[b]
    kc = k_ref[0]
    scores = jax.lax.dot_general(kc, u, (((1,), (1,)), ((), ())),
                                 preferred_element_type=jnp.float32)
    m = jnp.max(scores, axis=0, keepdims=True)
    e = jnp.exp(scores - m)
    s_ref[b] = jnp.sum(e, axis=0, keepdims=True)
    vc = v_ref[0]
    a_ref[b] = jax.lax.dot_general(e, vc, (((0,), (0,)), ((), ())),
                                   preferred_element_type=jnp.float32)

    @pl.when(b == B - 1)
    def _epilogue():
        r = 1.0 / s_ref[...]
        for h in range(HEADS):
            ah = a_ref[:, h, :] * r[:, 0, h:h + 1]
            ws = wv_ref[:, h * ATT:(h + 1) * ATT]
            upd_ref[:, h * ATT:(h + 1) * ATT] = jax.lax.dot_general(
                ah, ws, (((1,), (0,)), ((), ())),
                preferred_element_type=jnp.float32) + bv_ref[:, h * ATT:(h + 1) * ATT]
        out_ref[...] = jax.lax.dot_general(
            upd_ref[...], wo_ref[...], (((1,), (0,)), ((), ())),
            preferred_element_type=jnp.float32) + bo_ref[...]


def kernel(q, k, v, Wq, bq, Wk, bk, Wv, bv, Wo, bo):
    del bk
    B = q.shape[0]
    L_K = k.shape[1]
    full = lambda b: (0, 0)
    return pl.pallas_call(
        _mha_kernel,
        grid=(B,),
        in_specs=[
            pl.BlockSpec((B, HIDDEN), full),
            pl.BlockSpec((1, L_K, HIDDEN), lambda b: (b, 0, 0)),
            pl.BlockSpec((1, L_K, HIDDEN), lambda b: (b, 0, 0)),
            pl.BlockSpec((HIDDEN, HIDDEN), full),
            pl.BlockSpec((1, HIDDEN), full),
            pl.BlockSpec((HIDDEN, HIDDEN), full),
            pl.BlockSpec((HIDDEN, HIDDEN), full),
            pl.BlockSpec((1, HIDDEN), full),
            pl.BlockSpec((HIDDEN, HIDDEN), full),
            pl.BlockSpec((1, HIDDEN), full),
        ],
        out_specs=pl.BlockSpec((B, HIDDEN), full),
        out_shape=jax.ShapeDtypeStruct((B, HIDDEN), jnp.float32),
        scratch_shapes=[
            pltpu.VMEM((B, HEADS, HIDDEN), jnp.float32),
            pltpu.VMEM((B, HEADS, HIDDEN), jnp.float32),
            pltpu.VMEM((B, 1, HEADS), jnp.float32),
            pltpu.VMEM((B, HIDDEN), jnp.float32),
        ],
    )(q, k, v, Wq, bq.reshape(1, HIDDEN), Wk, Wv,
      bv.reshape(1, HIDDEN), Wo, bo.reshape(1, HIDDEN))


# speedup vs baseline: 1.5733x; 1.0056x over previous
import jax
import jax.numpy as jnp
from jax.experimental import pallas as pl
from jax.experimental.pallas import tpu as pltpu

HIDDEN = 1024
HEADS = 16
ATT = HIDDEN // HEADS
SCALE = ATT ** -0.5


def _mha_kernel(q_ref, k0_ref, k1_ref, v0_ref, v1_ref, wq_ref, bq_ref, wk_ref,
                wv_ref, bv_ref, wo_ref, bo_ref, out_ref, u_ref, a_ref, s_ref,
                upd_ref):
    b = pl.program_id(0)
    B = q_ref.shape[0]

    @pl.when(b == 0)
    def _prologue():
        qh = jax.lax.dot_general(q_ref[...], wq_ref[...],
                                 (((1,), (0,)), ((), ())),
                                 preferred_element_type=jnp.float32)
        qh = (qh + bq_ref[...]) * SCALE
        for h in range(HEADS):
            qs = qh[:, h * ATT:(h + 1) * ATT]
            ws = wk_ref[:, h * ATT:(h + 1) * ATT]
            u_ref[:, h, :] = jax.lax.dot_general(
                qs, ws, (((1,), (1,)), ((), ())),
                preferred_element_type=jnp.float32)

    u = u_ref[b]
    s0 = jax.lax.dot_general(k0_ref[0], u, (((1,), (1,)), ((), ())),
                             preferred_element_type=jnp.float32)
    s1 = jax.lax.dot_general(k1_ref[0], u, (((1,), (1,)), ((), ())),
                             preferred_element_type=jnp.float32)
    m = jnp.maximum(jnp.max(s0, axis=0, keepdims=True),
                    jnp.max(s1, axis=0, keepdims=True))
    e0 = jnp.exp(s0 - m)
    e1 = jnp.exp(s1 - m)
    s_ref[b] = (jnp.sum(e0, axis=0, keepdims=True)
                + jnp.sum(e1, axis=0, keepdims=True))
    a_ref[b] = (
        jax.lax.dot_general(e0, v0_ref[0], (((0,), (0,)), ((), ())),
                            preferred_element_type=jnp.float32)
        + jax.lax.dot_general(e1, v1_ref[0], (((0,), (0,)), ((), ())),
                              preferred_element_type=jnp.float32))

    @pl.when(b == B - 1)
    def _epilogue():
        r = 1.0 / s_ref[...]
        for h in range(HEADS):
            ah = a_ref[:, h, :] * r[:, 0, h:h + 1]
            ws = wv_ref[:, h * ATT:(h + 1) * ATT]
            upd_ref[:, h * ATT:(h + 1) * ATT] = jax.lax.dot_general(
                ah, ws, (((1,), (0,)), ((), ())),
                preferred_element_type=jnp.float32) + bv_ref[:, h * ATT:(h + 1) * ATT]
        out_ref[...] = jax.lax.dot_general(
            upd_ref[...], wo_ref[...], (((1,), (0,)), ((), ())),
            preferred_element_type=jnp.float32) + bo_ref[...]


def kernel(q, k, v, Wq, bq, Wk, bk, Wv, bv, Wo, bo):
    del bk
    B = q.shape[0]
    L_K = k.shape[1]
    L2 = L_K // 2
    full = lambda b: (0, 0)
    return pl.pallas_call(
        _mha_kernel,
        grid=(B,),
        in_specs=[
            pl.BlockSpec((B, HIDDEN), full),
            pl.BlockSpec((1, L2, HIDDEN), lambda b: (b, 0, 0)),
            pl.BlockSpec((1, L2, HIDDEN), lambda b: (b, 1, 0)),
            pl.BlockSpec((1, L2, HIDDEN), lambda b: (b, 0, 0)),
            pl.BlockSpec((1, L2, HIDDEN), lambda b: (b, 1, 0)),
            pl.BlockSpec((HIDDEN, HIDDEN), full),
            pl.BlockSpec((1, HIDDEN), full),
            pl.BlockSpec((HIDDEN, HIDDEN), full),
            pl.BlockSpec((HIDDEN, HIDDEN), full),
            pl.BlockSpec((1, HIDDEN), full),
            pl.BlockSpec((HIDDEN, HIDDEN), full),
            pl.BlockSpec((1, HIDDEN), full),
        ],
        out_specs=pl.BlockSpec((B, HIDDEN), full),
        out_shape=jax.ShapeDtypeStruct((B, HIDDEN), jnp.float32),
        scratch_shapes=[
            pltpu.VMEM((B, HEADS, HIDDEN), jnp.float32),
            pltpu.VMEM((B, HEADS, HIDDEN), jnp.float32),
            pltpu.VMEM((B, 1, HEADS), jnp.float32),
            pltpu.VMEM((B, HIDDEN), jnp.float32),
        ],
    )(q, k, k, v, v, Wq, bq.reshape(1, HIDDEN), Wk, Wv,
      bv.reshape(1, HIDDEN), Wo, bo.reshape(1, HIDDEN))
